# baseline (speedup 1.0000x reference)
import functools

import jax
import jax.numpy as jnp
from jax import lax
from jax.experimental import pallas as pl
from jax.experimental.pallas import tpu as pltpu
from jax.experimental.pallas import tpu_sc as plsc

FDIM = 256
NCON = 8192
NC = 2
NS = 16
NW = NC * NS
ROWS_PER_W = NCON // NW
L = 16


def _proj_body(w_ref, f_ref, b_ref, q_ref):
    q = jnp.sum(w_ref[...] * f_ref[...][None, :], axis=1) + b_ref[...]
    ss = jnp.sum(q * q)
    inv = 1.0 / jnp.maximum(jnp.sqrt(ss), 1e-12)
    q_ref[...] = q * inv


_proj = pl.pallas_call(
    _proj_body,
    out_shape=jax.ShapeDtypeStruct((FDIM,), jnp.float32),
)


_sc_mesh = plsc.VectorSubcoreMesh(core_axis_name="c", subcore_axis_name="s")


@functools.partial(
    pl.kernel,
    out_type=jax.ShapeDtypeStruct((NCON,), jnp.float32),
    mesh=_sc_mesh,
    compiler_params=pltpu.CompilerParams(needs_layout_passes=False),
    scratch_types=[
        pltpu.VMEM((FDIM,), jnp.float32),
        pltpu.VMEM((ROWS_PER_W, FDIM), jnp.float32),
        pltpu.VMEM((L, L), jnp.float32),
        pltpu.VMEM((ROWS_PER_W,), jnp.float32),
    ],
)
def _simkernel(q_hbm, proto_hbm, out_hbm, q_v, p_v, t_v, s_v):
    cid = lax.axis_index("c")
    sid = lax.axis_index("s")
    wid = sid * NC + cid
    base = wid * ROWS_PER_W
    pltpu.sync_copy(q_hbm, q_v)
    pltpu.sync_copy(proto_hbm.at[pl.ds(base, ROWS_PER_W)], p_v)

    qs = [q_v[pl.ds(L * j, L)] for j in range(FDIM // L)]
    lanes = lax.iota(jnp.int32, L)

    def group(g, carry):
        sims = jnp.zeros((L,), jnp.float32)
        for r in range(L):
            row = g * L + r
            acc = p_v[row, pl.ds(0, L)] * qs[0]
            for j in range(1, FDIM // L):
                acc = acc + p_v[row, pl.ds(L * j, L)] * qs[j]
            sims = jnp.where(lanes == r, jnp.sum(acc), sims)
        s_v[pl.ds(g * L, L)] = sims
        return carry

    lax.fori_loop(0, ROWS_PER_W // L, group, 0)
    pltpu.sync_copy(s_v, out_hbm.at[pl.ds(base, ROWS_PER_W)])


def _softmax_body(x_ref, act_ref, idx_ref):
    x = x_ref[...]
    t = x * 10.0
    m = jnp.max(t)
    e = jnp.exp(t - m)
    act_ref[...] = e / jnp.sum(e)
    mx = jnp.max(x)
    flat = lax.iota(jnp.int32, NCON)
    cand = jnp.where(x == mx, flat, jnp.int32(NCON))
    idx_ref[0] = jnp.min(cand)


_softmax = pl.pallas_call(
    _softmax_body,
    out_shape=(
        jax.ShapeDtypeStruct((NCON,), jnp.float32),
        jax.ShapeDtypeStruct((1,), jnp.int32),
    ),
    out_specs=(
        pl.BlockSpec(memory_space=pltpu.VMEM),
        pl.BlockSpec(memory_space=pltpu.SMEM),
    ),
)


def kernel(features, prototypes, W_vis, b_vis):
    q = _proj(W_vis, features, b_vis)
    sims = _simkernel(q, prototypes)
    act, idx = _softmax(sims)
    return act, idx[0]
